# Initial kernel scaffold; baseline (speedup 1.0000x reference)
import jax
import jax.numpy as jnp
import numpy as np
from jax import lax
from jax.experimental import pallas as pl
from jax.experimental.pallas import tpu as pltpu
from jax.experimental.pallas import tpu_sc as plsc

N = 10000
E = 320000
D = 128
DH = 64
NC, NS = 2, 16
CB = 128
CH = 157
EP = NS * CH * CB
RPT = 626
NP = NS * RPT
NRM_W = 640

_MESH = plsc.VectorSubcoreMesh(
    core_axis_name="c", subcore_axis_name="s", num_cores=NC, num_subcores=NS
)


def _hist_body(ei_ref, ones_ref, zeros_ref, hs_ref, hd_ref,
               idx_v, ones_v, hist_sh):
    c = lax.axis_index("c")
    s = lax.axis_index("s")
    rows = pl.ds(s * RPT, RPT)
    pltpu.sync_copy(zeros_ref.at[rows], hist_sh.at[rows])
    pltpu.sync_copy(ones_ref, ones_v)
    pltpu.sync_copy(ei_ref.at[c, s], idx_v)
    plsc.subcore_barrier()

    def chunk(j, carry):
        pltpu.sync_copy(ones_v, hist_sh.at[idx_v.at[j]], add=True)
        return carry

    lax.fori_loop(0, CH, chunk, 0)
    plsc.subcore_barrier()

    @pl.when(c == 0)
    def _():
        pltpu.sync_copy(hist_sh.at[rows], hs_ref.at[rows])

    @pl.when(c == 1)
    def _():
        pltpu.sync_copy(hist_sh.at[rows], hd_ref.at[rows])


_hist_kernel = pl.kernel(
    _hist_body,
    out_type=(
        jax.ShapeDtypeStruct((NP, 8), jnp.float32),
        jax.ShapeDtypeStruct((NP, 8), jnp.float32),
    ),
    mesh=_MESH,
    scratch_types=[
        pltpu.VMEM((CH, CB), jnp.int32),
        pltpu.VMEM((CB, 8), jnp.float32),
        pltpu.VMEM_SHARED((NP, 8), jnp.float32),
    ],
)


def _mm_body(x_ref, w_ref, b_ref, hs_ref, hd_ref,
             y0_ref, y1_ref, nrm_ref, bi0_ref, bi1_ref):
    outdeg = hs_ref[:, 0:1]
    nsrc = lax.rsqrt(jnp.maximum(outdeg, 1.0))
    h = x_ref[...] * nsrc
    y = jnp.dot(h, w_ref[...], preferred_element_type=jnp.float32)
    y0_ref[...] = y[:, :DH]
    y1_ref[...] = y[:, DH:]
    indeg = jnp.maximum(hd_ref[:, 0:1], 1.0)
    ndst = lax.rsqrt(indeg)
    nrm_ref[...] = jnp.broadcast_to(ndst, (NP, 8))
    sq = jnp.sqrt(indeg)
    b_row = b_ref[...].reshape(1, D)
    bi0_ref[...] = b_row[:, :DH] * sq
    bi1_ref[...] = b_row[:, DH:] * sq


_mm_kernel = pl.pallas_call(
    _mm_body,
    out_shape=(
        jax.ShapeDtypeStruct((NP, DH), jnp.float32),
        jax.ShapeDtypeStruct((NP, DH), jnp.float32),
        jax.ShapeDtypeStruct((NP, 8), jnp.float32),
        jax.ShapeDtypeStruct((NP, DH), jnp.float32),
        jax.ShapeDtypeStruct((NP, DH), jnp.float32),
    ),
)


def _mp_body(ei_ref, y0_ref, y1_ref, nrm_ref, bi0_ref, bi1_ref, out_ref,
             sidx, didx, rows_v, fin, nrm_s, acc_sh, sem):
    c = lax.axis_index("c")
    s = lax.axis_index("s")
    rows = pl.ds(s * RPT, RPT)

    @pl.when(c == 0)
    def _():
        pltpu.sync_copy(bi0_ref.at[rows], acc_sh.at[rows])

    @pl.when(c == 1)
    def _():
        pltpu.sync_copy(bi1_ref.at[rows], acc_sh.at[rows])

    pltpu.sync_copy(ei_ref.at[0, s], sidx)
    pltpu.sync_copy(ei_ref.at[1, s], didx)
    pltpu.sync_copy(nrm_ref.at[s], nrm_s)
    plsc.subcore_barrier()

    def run(y_ref):
        def chunk(j, carry):
            pltpu.async_copy(y_ref.at[sidx.at[j]], rows_v, sem).wait()
            pltpu.sync_copy(rows_v, acc_sh.at[didx.at[j]], add=True)
            return carry
        lax.fori_loop(0, CH, chunk, 0)

    @pl.when(c == 0)
    def _():
        run(y0_ref)

    @pl.when(c == 1)
    def _():
        run(y1_ref)

    plsc.subcore_barrier()
    pltpu.sync_copy(acc_sh.at[rows], fin)

    def frow(i, carry):
        sc = nrm_s[i]
        for k in range(DH // 16):
            col = pl.ds(k * 16, 16)
            fin[i, col] = fin[i, col] * sc
        return carry

    lax.fori_loop(0, RPT, frow, 0)
    pltpu.sync_copy(fin, out_ref.at[rows, pl.ds(c * DH, DH)])


_mp_kernel = pl.kernel(
    _mp_body,
    out_type=jax.ShapeDtypeStruct((NP, D), jnp.float32),
    mesh=_MESH,
    scratch_types=[
        pltpu.VMEM((CH, CB), jnp.int32),
        pltpu.VMEM((CH, CB), jnp.int32),
        pltpu.VMEM((CB, DH), jnp.float32),
        pltpu.VMEM((RPT, DH), jnp.float32),
        pltpu.SMEM((NRM_W,), jnp.float32),
        pltpu.VMEM_SHARED((NP, DH), jnp.float32),
        pltpu.SemaphoreType.DMA,
    ],
)

_ONES_ROWS = np.zeros((CB, 8), dtype=np.float32)
_ONES_ROWS[:, 0] = 1.0


@jax.jit
def kernel(x, edge_index, W, b):
    ei = edge_index.astype(jnp.int32)
    ei = jnp.pad(ei, ((0, 0), (0, EP - E)), constant_values=N)
    ei = ei.reshape(2, NS, CH, CB)
    x_pad = jnp.pad(x, ((0, NP - N), (0, 0)))
    ones_rows = jnp.asarray(_ONES_ROWS)
    zeros8 = jnp.zeros((NP, 8), jnp.float32)

    hs, hd = _hist_kernel(ei, ones_rows, zeros8)
    y0, y1, nrm8, bi0, bi1 = _mm_kernel(x_pad, W, b, hs, hd)
    nrm_r = jnp.pad(nrm8[:, 0].reshape(NS, RPT), ((0, 0), (0, NRM_W - RPT)))
    out_pad = _mp_kernel(ei, y0, y1, nrm_r, bi0, bi1)
    return out_pad[:N]


# speedup vs baseline: 6.2171x; 6.2171x over previous
import jax
import jax.numpy as jnp
from jax import lax
from jax.experimental import pallas as pl
from jax.experimental.pallas import tpu as pltpu
from jax.experimental.pallas import tpu_sc as plsc

N = 10000
E = 320000
D = 128
NC, NS = 2, 16
NW = NC * NS
CB = 128
CH = 79
EP = NW * CH * CB
RPT = 640
NP = NS * RPT

_MESH = plsc.VectorSubcoreMesh(
    core_axis_name="c", subcore_axis_name="s", num_cores=NC, num_subcores=NS
)


def _hist_body(ei_ref, hs_ref, hd_ref, idx_v, hist_v):
    c = lax.axis_index("c")
    s = lax.axis_index("s")

    def zero(i, carry):
        hist_v[pl.ds(i * 16, 16)] = jnp.zeros((16,), jnp.float32)
        return carry

    lax.fori_loop(0, NP // 16, zero, 0)
    pltpu.sync_copy(ei_ref.at[c, s], idx_v)
    ones = jnp.ones((16,), jnp.float32)

    def chunk(j, carry):
        for k in range(CB // 16):
            idx16 = idx_v[j, pl.ds(k * 16, 16)]
            plsc.addupdate_scatter(hist_v, [idx16], ones)
        return carry

    lax.fori_loop(0, 2 * CH, chunk, 0)

    @pl.when(c == 0)
    def _():
        pltpu.sync_copy(hist_v, hs_ref.at[s])

    @pl.when(c == 1)
    def _():
        pltpu.sync_copy(hist_v, hd_ref.at[s])


_hist_kernel = pl.kernel(
    _hist_body,
    out_type=(
        jax.ShapeDtypeStruct((NS, NP), jnp.float32),
        jax.ShapeDtypeStruct((NS, NP), jnp.float32),
    ),
    mesh=_MESH,
    scratch_types=[
        pltpu.VMEM((2 * CH, CB), jnp.int32),
        pltpu.VMEM((NP,), jnp.float32),
    ],
    compiler_params=pltpu.CompilerParams(needs_layout_passes=False),
)


def _mm_body(x_ref, w_ref, hs_ref, hd_ref, y_ref, nrm_ref):
    ones_col = jnp.ones((NS, 1), jnp.float32)
    dn = (((0,), (0,)), ((), ()))
    outdeg = lax.dot_general(hs_ref[...], ones_col, dn,
                             preferred_element_type=jnp.float32)
    nsrc = lax.rsqrt(jnp.maximum(outdeg, 1.0))
    h = x_ref[...] * nsrc
    y_ref[...] = jnp.dot(h, w_ref[...], preferred_element_type=jnp.float32)
    indeg = lax.dot_general(hd_ref[...], ones_col, dn,
                            preferred_element_type=jnp.float32)
    ndst = lax.rsqrt(jnp.maximum(indeg, 1.0))
    nrm_ref[...] = jnp.broadcast_to(ndst, (NP, 8))


_mm_kernel = pl.pallas_call(
    _mm_body,
    out_shape=(
        jax.ShapeDtypeStruct((NP, D), jnp.float32),
        jax.ShapeDtypeStruct((NP, 8), jnp.float32),
    ),
)


def _mp_body(ei_ref, y_ref, zeros_ref, p0_ref, p1_ref,
             sidx, didx, rows_v, acc_sh, sem):
    c = lax.axis_index("c")
    s = lax.axis_index("s")
    q = c * NS + s
    rows = pl.ds(s * RPT, RPT)
    pltpu.sync_copy(zeros_ref.at[rows], acc_sh.at[rows])
    pltpu.sync_copy(ei_ref.at[0, q], sidx)
    pltpu.sync_copy(ei_ref.at[1, q], didx)
    plsc.subcore_barrier()

    def chunk(j, carry):
        pltpu.async_copy(y_ref.at[sidx.at[j]], rows_v, sem).wait()
        pltpu.sync_copy(rows_v, acc_sh.at[didx.at[j]], add=True)
        return carry

    lax.fori_loop(0, CH, chunk, 0)
    plsc.subcore_barrier()

    @pl.when(c == 0)
    def _():
        pltpu.sync_copy(acc_sh.at[rows], p0_ref.at[rows])

    @pl.when(c == 1)
    def _():
        pltpu.sync_copy(acc_sh.at[rows], p1_ref.at[rows])


_mp_kernel = pl.kernel(
    _mp_body,
    out_type=(
        jax.ShapeDtypeStruct((NP, D), jnp.float32),
        jax.ShapeDtypeStruct((NP, D), jnp.float32),
    ),
    mesh=_MESH,
    scratch_types=[
        pltpu.VMEM((CH, CB), jnp.int32),
        pltpu.VMEM((CH, CB), jnp.int32),
        pltpu.VMEM((CB, D), jnp.float32),
        pltpu.VMEM_SHARED((NP, D), jnp.float32),
        pltpu.SemaphoreType.DMA,
    ],
)


def _fin_body(p0_ref, p1_ref, nrm_ref, b_ref, out_ref):
    nrm = nrm_ref[:, 0:1]
    b_row = b_ref[...].reshape(1, D)
    out_ref[...] = (p0_ref[...] + p1_ref[...]) * nrm + b_row


_fin_kernel = pl.pallas_call(
    _fin_body,
    out_shape=jax.ShapeDtypeStruct((NP, D), jnp.float32),
)


@jax.jit
def kernel(x, edge_index, W, b):
    ei = edge_index.astype(jnp.int32)
    ei = jnp.pad(ei, ((0, 0), (0, EP - E)), constant_values=N)
    ei = ei.reshape(2, NW, CH, CB)
    x_pad = jnp.pad(x, ((0, NP - N), (0, 0)))
    zeros = jnp.zeros((NP, D), jnp.float32)

    ei_k1 = ei.reshape(2, NS, 2 * CH, CB)
    hs, hd = _hist_kernel(ei_k1)
    y, nrm8 = _mm_kernel(x_pad, W, hs, hd)
    p0, p1 = _mp_kernel(ei, y, zeros)
    out_pad = _fin_kernel(p0, p1, nrm8, b)
    return out_pad[:N]


# speedup vs baseline: 14.0204x; 2.2551x over previous
import jax
import jax.numpy as jnp
from jax import lax
from jax.experimental import pallas as pl
from jax.experimental.pallas import tpu as pltpu
from jax.experimental.pallas import tpu_sc as plsc

N = 10000
E = 320000
D = 128
NC, NS = 2, 16
NW = NC * NS
CB = 128
CH = 80
HCH = 40
EP = NW * CH * CB
RPT = 640
NP = NS * RPT

_MESH = plsc.VectorSubcoreMesh(
    core_axis_name="c", subcore_axis_name="s", num_cores=NC, num_subcores=NS
)


def _hist_body(ei_ref, hs_ref, hd_ref, idx_v, hist_v):
    c = lax.axis_index("c")
    s = lax.axis_index("s")

    def zero(i, carry):
        hist_v[pl.ds(i * 16, 16)] = jnp.zeros((16,), jnp.float32)
        return carry

    lax.fori_loop(0, NP // 16, zero, 0)
    pltpu.sync_copy(ei_ref.at[c, s], idx_v)
    ones = jnp.ones((16,), jnp.float32)

    def chunk(j, carry):
        for k in range(CB // 16):
            idx16 = idx_v[j, pl.ds(k * 16, 16)]
            plsc.addupdate_scatter(hist_v, [idx16], ones)
        return carry

    lax.fori_loop(0, 2 * CH, chunk, 0)

    @pl.when(c == 0)
    def _():
        pltpu.sync_copy(hist_v, hs_ref.at[s])

    @pl.when(c == 1)
    def _():
        pltpu.sync_copy(hist_v, hd_ref.at[s])


_hist_kernel = pl.kernel(
    _hist_body,
    out_type=(
        jax.ShapeDtypeStruct((NS, NP), jnp.float32),
        jax.ShapeDtypeStruct((NS, NP), jnp.float32),
    ),
    mesh=_MESH,
    scratch_types=[
        pltpu.VMEM((2 * CH, CB), jnp.int32),
        pltpu.VMEM((NP,), jnp.float32),
    ],
    compiler_params=pltpu.CompilerParams(needs_layout_passes=False),
)


def _mm_body(x_ref, w_ref, hs_ref, hd_ref, y_ref, nrm_ref):
    ones_col = jnp.ones((NS, 1), jnp.float32)
    dn = (((0,), (0,)), ((), ()))
    outdeg = lax.dot_general(hs_ref[...], ones_col, dn,
                             preferred_element_type=jnp.float32)
    nsrc = lax.rsqrt(jnp.maximum(outdeg, 1.0))
    h = x_ref[...] * nsrc
    y_ref[...] = jnp.dot(h, w_ref[...], preferred_element_type=jnp.float32)
    indeg = lax.dot_general(hd_ref[...], ones_col, dn,
                            preferred_element_type=jnp.float32)
    ndst = lax.rsqrt(jnp.maximum(indeg, 1.0))
    nrm_ref[...] = jnp.broadcast_to(ndst, (NP, 8))


_mm_kernel = pl.pallas_call(
    _mm_body,
    out_shape=(
        jax.ShapeDtypeStruct((NP, D), jnp.float32),
        jax.ShapeDtypeStruct((NP, 8), jnp.float32),
    ),
)


def _mp_body(ei_ref, y_ref, zeros_ref, p0_ref, p1_ref,
             sidx, didx, rows_a, rows_b, acc_sh, sem):
    c = lax.axis_index("c")
    s = lax.axis_index("s")
    q = c * NS + s
    rows = pl.ds(s * RPT, RPT)
    pltpu.sync_copy(zeros_ref.at[rows], acc_sh.at[rows])
    plsc.subcore_barrier()

    bufs = (rows_a, rows_b)
    for h in range(CH // HCH):
        pltpu.sync_copy(ei_ref.at[0, q, pl.ds(h * HCH, HCH)], sidx)
        pltpu.sync_copy(ei_ref.at[1, q, pl.ds(h * HCH, HCH)], didx)
        pltpu.async_copy(y_ref.at[sidx.at[0]], rows_a, sem)

        def pair(g, carry):
            for bsel in range(2):
                j = 2 * g + bsel
                buf = bufs[bsel]

                @pl.when(j + 1 < HCH)
                def _():
                    pltpu.async_copy(
                        y_ref.at[sidx.at[j + 1]], bufs[1 - bsel], sem)

                pltpu.make_async_copy(y_ref.at[sidx.at[j]], buf, sem).wait()
                pltpu.sync_copy(buf, acc_sh.at[didx.at[j]], add=True)
            return carry

        lax.fori_loop(0, HCH // 2, pair, 0)
    plsc.subcore_barrier()

    @pl.when(c == 0)
    def _():
        pltpu.sync_copy(acc_sh.at[rows], p0_ref.at[rows])

    @pl.when(c == 1)
    def _():
        pltpu.sync_copy(acc_sh.at[rows], p1_ref.at[rows])


_mp_kernel = pl.kernel(
    _mp_body,
    out_type=(
        jax.ShapeDtypeStruct((NP, D), jnp.float32),
        jax.ShapeDtypeStruct((NP, D), jnp.float32),
    ),
    mesh=_MESH,
    scratch_types=[
        pltpu.VMEM((HCH, CB), jnp.int32),
        pltpu.VMEM((HCH, CB), jnp.int32),
        pltpu.VMEM((CB, D), jnp.float32),
        pltpu.VMEM((CB, D), jnp.float32),
        pltpu.VMEM_SHARED((NP, D), jnp.float32),
        pltpu.SemaphoreType.DMA,
    ],
)


def _fin_body(p0_ref, p1_ref, nrm_ref, b_ref, out_ref):
    nrm = nrm_ref[:, 0:1]
    b_row = b_ref[...].reshape(1, D)
    out_ref[...] = (p0_ref[...] + p1_ref[...]) * nrm + b_row


_fin_kernel = pl.pallas_call(
    _fin_body,
    out_shape=jax.ShapeDtypeStruct((NP, D), jnp.float32),
)


@jax.jit
def kernel(x, edge_index, W, b):
    ei = edge_index.astype(jnp.int32)
    pad_idx = N + jnp.arange(EP - E, dtype=jnp.int32) % (NP - N)
    pad_blk = jnp.broadcast_to(pad_idx, (2, EP - E))
    ei = jnp.concatenate([ei, pad_blk], axis=1)
    ei = ei.reshape(2, NW, CH, CB)
    x_pad = jnp.pad(x, ((0, NP - N), (0, 0)))
    zeros = jnp.zeros((NP, D), jnp.float32)

    ei_k1 = ei.reshape(2, NS, 2 * CH, CB)
    hs, hd = _hist_kernel(ei_k1)
    y, nrm8 = _mm_kernel(x_pad, W, hs, hd)
    p0, p1 = _mp_kernel(ei, y, zeros)
    out_pad = _fin_kernel(p0, p1, nrm8, b)
    return out_pad[:N]
